# baseline (speedup 1.0000x reference)
import jax
import jax.numpy as jnp
from jax import lax
from jax.experimental import pallas as pl
from jax.experimental.pallas import tpu as pltpu
from jax.experimental.pallas import tpu_sc as plsc

N = 50000
E = 800000
D = 64

NCORES = 2
NSUB = 16
H = N // NCORES
RPT = 1568
HPAD = NSUB * RPT
TRASH = HPAD
ACC_ROWS = HPAD + 8
K = 80
EPT = E // NSUB
NCHUNK = EPT // K

BLK = 2000
NBLK = N // BLK


G = 80
NSLOT = 3
RPB = 6
BCH = NSLOT * RPB * G
NCHK = BCH // G
NBATCH = 35
SPAN = NBATCH * BCH
REG = NSLOT * G
CBUF = BCH + 608
STRIP = CBUF + 16
DUMP = BCH + 592
PKSH = 32768
PKTRASH = TRASH


def _make_seg_kernel():
    mesh = plsc.VectorSubcoreMesh(core_axis_name="c", subcore_axis_name="s")

    def body(h_hbm, src_hbm, dst_hbm, zeros_hbm, padfill_hbm,
             agg_hbm,
             acc, stg, ibS, ibD, cPK, cntb2, csm, pidx, sidx, lgrps, rows,
             semIS, semID, semC, semG, semU, semX):
        c = lax.axis_index("c")
        s = lax.axis_index("s")
        base = c * H
        tbase = s * SPAN
        iota = lax.iota(jnp.int32, 16)

        pltpu.sync_copy(zeros_hbm, acc.at[pl.ds(s * RPT, RPT)])

        def issue_batch_loads(p, b):
            off = tbase + b * BCH
            pltpu.async_copy(src_hbm.at[pl.ds(off, BCH)], ibS[p], semIS[p])
            pltpu.async_copy(dst_hbm.at[pl.ds(off, BCH)], ibD[p], semID[p])

        def prep_batch(p):
            pltpu.make_async_copy(src_hbm.at[pl.ds(0, BCH)], ibS[p],
                                  semIS[p]).wait()
            pltpu.make_async_copy(src_hbm.at[pl.ds(0, BCH)], ibD[p],
                                  semID[p]).wait()
            pltpu.sync_copy(padfill_hbm, stg.at[pl.ds(s * STRIP, CBUF)])

            posv = jnp.zeros((16,), jnp.int32)
            for bb in range(NCHK):
                def prep16(g2, pv, bb=bb):
                    sl = pl.ds(bb * G + g2 * 16, 16)
                    s16 = ibS[p][sl]
                    loc = ibD[p][sl] - base
                    ok = (loc >= 0) & (loc < H)
                    pr = iota + 1
                    tgt = s * STRIP + jnp.where(ok, pv + pr - 1,
                                                DUMP + iota)
                    ibS[p][sl] = jnp.where(ok, s16 * PKSH + loc, PKTRASH)
                    pidx[bb][pl.ds(g2 * 16, 16)] = tgt
                    return pv + pr[jnp.full((16,), 15, jnp.int32)]

                posv = lax.fori_loop(0, G // 16, prep16, posv)
                pltpu.async_copy(ibS[p].at[pl.ds(bb * G, G)],
                                 stg.at[pidx[bb]], semC[p])
            cntb2[p][...] = posv
            pltpu.async_copy(cntb2[p], stg.at[pl.ds(s * STRIP + CBUF, 16)],
                             semX[p])
            for _ in range(NCHK):
                pltpu.make_async_copy(padfill_hbm.at[pl.ds(0, G)],
                                      cPK[p].at[pl.ds(0, G)],
                                      semC[p]).wait()
            pltpu.sync_copy(stg.at[pl.ds(s * STRIP, CBUF)], cPK[p])
            pltpu.make_async_copy(cntb2[p],
                                  stg.at[pl.ds(s * STRIP + CBUF, 16)],
                                  semX[p]).wait()
            pltpu.sync_copy(stg.at[pl.ds(s * STRIP + CBUF, 16)], csm[p])

        def wait_scatter(k):
            pltpu.make_async_copy(h_hbm.at[pl.ds(0, G)], rows[k],
                                  semU[k]).wait()

        def issue_gather(cp, k, gl):
            for i in range(G // 16):
                sidx[k][pl.ds(i * 16, 16)] = lax.shift_right_logical(
                    cp[pl.ds(gl * G + i * 16, 16)], 15)
            pltpu.async_copy(h_hbm.at[sidx[k]], rows[k], semG[k])

        def slot_cycle(cp, k, gl):
            pltpu.make_async_copy(h_hbm.at[pl.ds(0, G)], rows[k],
                                  semG[k]).wait()
            for i in range(G // 16):
                lgrps[k][pl.ds(i * 16, 16)] = (
                    cp[pl.ds(gl * G + i * 16, 16)] & (PKSH - 1))
            pltpu.async_copy(rows[k], acc.at[lgrps[k]], semU[k], add=True)

        def prime(p):
            cp = cPK[p]
            cnt = csm[p][0]

            @pl.when(cnt > 0)
            def _():
                for k in range(NSLOT):
                    issue_gather(cp, k, k)

        def run_rounds(p):
            cp = cPK[p]
            cnt = csm[p][0]
            for r in range(RPB):
                @pl.when(cnt > r * REG)
                def _(r=r):
                    for k in range(NSLOT):
                        slot_cycle(cp, k, r * NSLOT + k)
                    for k in range(NSLOT):
                        wait_scatter(k)

                    @pl.when(cnt > (r + 1) * REG)
                    def _():
                        for k in range(NSLOT):
                            issue_gather(cp, k, (r + 1) * NSLOT + k)

        issue_batch_loads(0, 0)
        issue_batch_loads(1, 1)
        prep_batch(0)
        prime(0)
        plsc.subcore_barrier()

        def batch(b, carry):
            @pl.when(b % 2 == 0)
            def _():
                run_rounds(0)

                @pl.when(b + 2 < NBATCH)
                def _():
                    issue_batch_loads(0, b + 2)

                @pl.when(b + 1 < NBATCH)
                def _():
                    prep_batch(1)
                    prime(1)

            @pl.when(b % 2 == 1)
            def _():
                run_rounds(1)

                @pl.when(b + 2 < NBATCH)
                def _():
                    issue_batch_loads(1, b + 2)

                @pl.when(b + 1 < NBATCH)
                def _():
                    prep_batch(0)
                    prime(0)

            return carry

        lax.fori_loop(0, NBATCH, batch, 0)
        plsc.subcore_barrier()

        loff = jnp.minimum(s * RPT, H - RPT)
        pltpu.sync_copy(acc.at[pl.ds(loff, RPT)],
                        agg_hbm.at[pl.ds(base + loff, RPT)])

    return pl.kernel(
        body,
        out_type=jax.ShapeDtypeStruct((N, D), jnp.float32),
        mesh=mesh,
        compiler_params=pltpu.CompilerParams(use_tc_tiling_on_sc=False),
        scratch_types=[
            pltpu.VMEM_SHARED((ACC_ROWS, D), jnp.float32),
            pltpu.VMEM_SHARED((NSUB * STRIP,), jnp.int32),
            [pltpu.VMEM((BCH,), jnp.int32) for _ in range(2)],
            [pltpu.VMEM((BCH,), jnp.int32) for _ in range(2)],
            [pltpu.VMEM((CBUF,), jnp.int32) for _ in range(2)],
            [pltpu.VMEM((16,), jnp.int32) for _ in range(2)],
            [pltpu.SMEM((16,), jnp.int32) for _ in range(2)],
            [pltpu.VMEM((G,), jnp.int32) for _ in range(NCHK)],
            [pltpu.VMEM((G,), jnp.int32) for _ in range(NSLOT)],
            [pltpu.VMEM((G,), jnp.int32) for _ in range(NSLOT)],
            [pltpu.VMEM((G, D), jnp.float32) for _ in range(NSLOT)],
            [pltpu.SemaphoreType.DMA for _ in range(2)],
            [pltpu.SemaphoreType.DMA for _ in range(2)],
            [pltpu.SemaphoreType.DMA for _ in range(2)],
            [pltpu.SemaphoreType.DMA for _ in range(NSLOT)],
            [pltpu.SemaphoreType.DMA for _ in range(NSLOT)],
            [pltpu.SemaphoreType.DMA for _ in range(2)],
        ],
    )


_seg_kernel = _make_seg_kernel()


def _embed_body(x_ref, y_ref, we_ref, b_ref, o_ref):
    o_ref[...] = (x_ref[...] * we_ref[0:1, :]
                  + y_ref[...] * we_ref[1:2, :] + b_ref[...])


def _embed(x, y, W_embed, b):
    return pl.pallas_call(
        _embed_body,
        grid=(NBLK,),
        in_specs=[
            pl.BlockSpec((BLK, 1), lambda i: (i, 0)),
            pl.BlockSpec((BLK, 1), lambda i: (i, 0)),
            pl.BlockSpec((2, D), lambda i: (0, 0)),
            pl.BlockSpec((1, D), lambda i: (0, 0)),
        ],
        out_specs=pl.BlockSpec((BLK, D), lambda i: (i, 0)),
        out_shape=jax.ShapeDtypeStruct((N, D), jnp.float32),
    )(x, y, W_embed, b)


def _update_body(h_ref, agg_ref, w_ref, b_ref, o_ref):
    z = jnp.dot(agg_ref[...], w_ref[...],
                preferred_element_type=jnp.float32) + b_ref[...]
    o_ref[...] = h_ref[...] + jnp.maximum(z, 0.0)


def _update(h, agg, W, b):
    return pl.pallas_call(
        _update_body,
        grid=(NBLK,),
        in_specs=[
            pl.BlockSpec((BLK, D), lambda i: (i, 0)),
            pl.BlockSpec((BLK, D), lambda i: (i, 0)),
            pl.BlockSpec((D, D), lambda i: (0, 0)),
            pl.BlockSpec((1, D), lambda i: (0, 0)),
        ],
        out_specs=pl.BlockSpec((BLK, D), lambda i: (i, 0)),
        out_shape=jax.ShapeDtypeStruct((N, D), jnp.float32),
    )(h, agg, W, b)


def _score_body(h_ref, w_ref, nt_ref, masked_ref, lse_ref, m_s, s_s):
    i = pl.program_id(0)
    sc = jnp.dot(h_ref[...], w_ref[...], preferred_element_type=jnp.float32)
    masked = jnp.where(nt_ref[...] == 2, sc, jnp.float32(-1e9))
    masked_ref[...] = masked

    @pl.when(i == 0)
    def _():
        m_s[0] = jnp.float32(-1e30)
        s_s[0] = jnp.float32(0.0)

    m_old = m_s[0]
    m_blk = jnp.max(masked)
    m_new = jnp.maximum(m_old, m_blk)
    s_s[0] = (s_s[0] * jnp.exp(m_old - m_new)
              + jnp.sum(jnp.exp(masked - m_new)))
    m_s[0] = m_new

    @pl.when(i == NBLK - 1)
    def _():
        lse_ref[...] = jnp.full((1, 1), m_s[0] + jnp.log(s_s[0]), jnp.float32)


def _score(h, w_score2d, node_type2d):
    return pl.pallas_call(
        _score_body,
        grid=(NBLK,),
        in_specs=[
            pl.BlockSpec((BLK, D), lambda i: (i, 0)),
            pl.BlockSpec((D, 1), lambda i: (0, 0)),
            pl.BlockSpec((BLK, 1), lambda i: (i, 0)),
        ],
        out_specs=[
            pl.BlockSpec((BLK, 1), lambda i: (i, 0)),
            pl.BlockSpec((1, 1), lambda i: (0, 0)),
        ],
        out_shape=[
            jax.ShapeDtypeStruct((N, 1), jnp.float32),
            jax.ShapeDtypeStruct((1, 1), jnp.float32),
        ],
        scratch_shapes=[
            pltpu.SMEM((1,), jnp.float32),
            pltpu.SMEM((1,), jnp.float32),
        ],
    )(h, w_score2d, node_type2d)


def _finish_body(masked_ref, lse_ref, o_ref):
    o_ref[...] = masked_ref[...] - lse_ref[0, 0]


def _finish(masked, lse):
    return pl.pallas_call(
        _finish_body,
        grid=(NBLK,),
        in_specs=[
            pl.BlockSpec((BLK, 1), lambda i: (i, 0)),
            pl.BlockSpec((1, 1), lambda i: (0, 0)),
        ],
        out_specs=pl.BlockSpec((BLK, 1), lambda i: (i, 0)),
        out_shape=jax.ShapeDtypeStruct((N, 1), jnp.float32),
    )(masked, lse)


@jax.jit
def kernel(coord, W_embed, b_embed, W0, b0, W1, b1, W2, b2, w_score,
           edge_index, node_type):
    src = jnp.pad(edge_index[0].reshape(NSUB, EPT),
                  ((0, 0), (0, SPAN - EPT))).reshape(-1)
    dst = jnp.pad(edge_index[1].reshape(NSUB, EPT),
                  ((0, 0), (0, SPAN - EPT)),
                  constant_values=-1).reshape(-1)
    zeros = jnp.zeros((RPT, D), jnp.float32)
    padfill = jnp.full((CBUF,), PKTRASH, jnp.int32)
    x = coord[:, 0:1]
    y = coord[:, 1:2]

    h = _embed(x, y, W_embed, b_embed.reshape(1, D))
    for (W, b) in ((W0, b0), (W1, b1), (W2, b2)):
        agg = _seg_kernel(h, src, dst, zeros, padfill)
        h = _update(h, agg, W, b.reshape(1, D))

    masked, lse = _score(h, w_score.reshape(D, 1), node_type.reshape(N, 1))
    out = _finish(masked, lse)
    return out.reshape(N)


# speedup vs baseline: 17.3670x; 17.3670x over previous
import jax
import jax.numpy as jnp
from jax import lax
from jax.experimental import pallas as pl
from jax.experimental.pallas import tpu as pltpu
from jax.experimental.pallas import tpu_sc as plsc

N = 50000
E = 800000
D = 64

NCORES = 2
NSUB = 16
H = N // NCORES
RPT = 1568
HPAD = NSUB * RPT
TRASH = HPAD
ACC_ROWS = HPAD + 8
K = 80
EPT = E // NSUB
NCHUNK = EPT // K

BLK = 2000
NBLK = N // BLK


G = 112
NSLOT = 3
RPB = 6
BGRP = NSLOT * RPB
BCH = BGRP * G
NBATCH = 25
SPAN = NBATCH * BCH


def _make_seg_kernel():
    mesh = plsc.VectorSubcoreMesh(core_axis_name="c", subcore_axis_name="s")

    def body(h_hbm, src_hbm, dst_hbm, zeros_hbm, agg_hbm,
             acc, ibS, ibD, lgrps, rows, semIS, semID, semG, semU):
        c = lax.axis_index("c")
        s = lax.axis_index("s")
        base = c * H
        tbase = s * SPAN

        pltpu.sync_copy(zeros_hbm, acc.at[pl.ds(s * RPT, RPT)])

        def issue_batch_loads(p, b):
            off = tbase + b * BCH
            pltpu.async_copy(src_hbm.at[pl.ds(off, BCH)], ibS[p], semIS[p])
            pltpu.async_copy(dst_hbm.at[pl.ds(off, BCH)], ibD[p], semID[p])

        def wait_scatter(k):
            pltpu.make_async_copy(h_hbm.at[pl.ds(0, G)], rows[k],
                                  semU[k]).wait()

        def issue_gather(bS, k, gl):
            wait_scatter(k)
            pltpu.async_copy(h_hbm.at[bS.at[pl.ds(gl * G, G)]],
                             rows[k], semG[k])

        def slot_cycle(bD, k, gl):
            pltpu.make_async_copy(h_hbm.at[pl.ds(0, G)], rows[k],
                                  semG[k]).wait()
            for i in range(G // 16):
                loc = bD[pl.ds(gl * G + i * 16, 16)] - base
                ok = (loc >= 0) & (loc < H)
                lgrps[k][pl.ds(i * 16, 16)] = jnp.where(ok, loc, TRASH)
            pltpu.async_copy(rows[k], acc.at[lgrps[k]], semU[k], add=True)

        def run_batch(p):
            bS, bD = ibS[p], ibD[p]
            pltpu.make_async_copy(src_hbm.at[pl.ds(0, BCH)], bS,
                                  semIS[p]).wait()
            pltpu.make_async_copy(src_hbm.at[pl.ds(0, BCH)], bD,
                                  semID[p]).wait()
            for k in range(NSLOT):
                issue_gather(bS, k, k)

            def round_(rr, carry):
                for k in range(NSLOT):
                    slot_cycle(bD, k, rr * NSLOT + k)
                for k in range(NSLOT):
                    issue_gather(bS, k, (rr + 1) * NSLOT + k)
                return carry

            lax.fori_loop(0, RPB - 1, round_, 0)
            for k in range(NSLOT):
                slot_cycle(bD, k, (RPB - 1) * NSLOT + k)

        for k in range(NSLOT):
            for i in range(G // 16):
                lgrps[k][pl.ds(i * 16, 16)] = jnp.full((16,), TRASH,
                                                       jnp.int32)
            pltpu.async_copy(rows[k], acc.at[lgrps[k]], semU[k], add=True)

        issue_batch_loads(0, 0)
        issue_batch_loads(1, 1)
        plsc.subcore_barrier()

        def batch(b, carry):
            @pl.when(b % 2 == 0)
            def _():
                run_batch(0)

            @pl.when(b % 2 == 1)
            def _():
                run_batch(1)

            @pl.when(b + 2 < NBATCH)
            def _():
                @pl.when(b % 2 == 0)
                def _():
                    issue_batch_loads(0, b + 2)

                @pl.when(b % 2 == 1)
                def _():
                    issue_batch_loads(1, b + 2)

            return carry

        lax.fori_loop(0, NBATCH, batch, 0)

        for k in range(NSLOT):
            wait_scatter(k)
        plsc.subcore_barrier()

        loff = jnp.minimum(s * RPT, H - RPT)
        pltpu.sync_copy(acc.at[pl.ds(loff, RPT)],
                        agg_hbm.at[pl.ds(base + loff, RPT)])

    return pl.kernel(
        body,
        out_type=jax.ShapeDtypeStruct((N, D), jnp.float32),
        mesh=mesh,
        compiler_params=pltpu.CompilerParams(use_tc_tiling_on_sc=False),
        scratch_types=[
            pltpu.VMEM_SHARED((ACC_ROWS, D), jnp.float32),
            [pltpu.VMEM((BCH,), jnp.int32) for _ in range(2)],
            [pltpu.VMEM((BCH,), jnp.int32) for _ in range(2)],
            [pltpu.VMEM((G,), jnp.int32) for _ in range(NSLOT)],
            [pltpu.VMEM((G, D), jnp.float32) for _ in range(NSLOT)],
            [pltpu.SemaphoreType.DMA for _ in range(2)],
            [pltpu.SemaphoreType.DMA for _ in range(2)],
            [pltpu.SemaphoreType.DMA for _ in range(NSLOT)],
            [pltpu.SemaphoreType.DMA for _ in range(NSLOT)],
        ],
    )


_seg_kernel = _make_seg_kernel()


def _embed_body(x_ref, y_ref, we_ref, b_ref, o_ref):
    o_ref[...] = (x_ref[...] * we_ref[0:1, :]
                  + y_ref[...] * we_ref[1:2, :] + b_ref[...])


def _embed(x, y, W_embed, b):
    return pl.pallas_call(
        _embed_body,
        grid=(NBLK,),
        in_specs=[
            pl.BlockSpec((BLK, 1), lambda i: (i, 0)),
            pl.BlockSpec((BLK, 1), lambda i: (i, 0)),
            pl.BlockSpec((2, D), lambda i: (0, 0)),
            pl.BlockSpec((1, D), lambda i: (0, 0)),
        ],
        out_specs=pl.BlockSpec((BLK, D), lambda i: (i, 0)),
        out_shape=jax.ShapeDtypeStruct((N, D), jnp.float32),
    )(x, y, W_embed, b)


def _update_body(h_ref, agg_ref, w_ref, b_ref, o_ref):
    z = jnp.dot(agg_ref[...], w_ref[...],
                preferred_element_type=jnp.float32) + b_ref[...]
    o_ref[...] = h_ref[...] + jnp.maximum(z, 0.0)


def _update(h, agg, W, b):
    return pl.pallas_call(
        _update_body,
        grid=(NBLK,),
        in_specs=[
            pl.BlockSpec((BLK, D), lambda i: (i, 0)),
            pl.BlockSpec((BLK, D), lambda i: (i, 0)),
            pl.BlockSpec((D, D), lambda i: (0, 0)),
            pl.BlockSpec((1, D), lambda i: (0, 0)),
        ],
        out_specs=pl.BlockSpec((BLK, D), lambda i: (i, 0)),
        out_shape=jax.ShapeDtypeStruct((N, D), jnp.float32),
    )(h, agg, W, b)


def _score_body(h_ref, w_ref, nt_ref, o_ref, msk_s, m_s, s_s):
    i = pl.program_id(0)

    @pl.when(i < NBLK)
    def _():
        sc = jnp.dot(h_ref[...], w_ref[...],
                     preferred_element_type=jnp.float32)
        masked = jnp.where(nt_ref[...] == 2, sc, jnp.float32(-1e9))
        msk_s[pl.ds((i % NBLK) * BLK, BLK), :] = masked

        @pl.when(i == 0)
        def _():
            m_s[0] = jnp.float32(-1e30)
            s_s[0] = jnp.float32(0.0)

        m_old = m_s[0]
        m_new = jnp.maximum(m_old, jnp.max(masked))
        s_s[0] = (s_s[0] * jnp.exp(m_old - m_new)
                  + jnp.sum(jnp.exp(masked - m_new)))
        m_s[0] = m_new

    @pl.when(i >= NBLK)
    def _():
        lse = m_s[0] + jnp.log(s_s[0])
        o_ref[...] = msk_s[pl.ds((i % NBLK) * BLK, BLK), :] - lse


def _score(h, w_score2d, node_type2d):
    return pl.pallas_call(
        _score_body,
        grid=(2 * NBLK,),
        in_specs=[
            pl.BlockSpec((BLK, D), lambda i: (i % NBLK, 0)),
            pl.BlockSpec((D, 1), lambda i: (0, 0)),
            pl.BlockSpec((BLK, 1), lambda i: (i % NBLK, 0)),
        ],
        out_specs=pl.BlockSpec((BLK, 1), lambda i: (i % NBLK, 0)),
        out_shape=jax.ShapeDtypeStruct((N, 1), jnp.float32),
        scratch_shapes=[
            pltpu.VMEM((N, 1), jnp.float32),
            pltpu.SMEM((1,), jnp.float32),
            pltpu.SMEM((1,), jnp.float32),
        ],
    )(h, w_score2d, node_type2d)


@jax.jit
def kernel(coord, W_embed, b_embed, W0, b0, W1, b1, W2, b2, w_score,
           edge_index, node_type):
    src = jnp.pad(edge_index[0].reshape(NSUB, EPT),
                  ((0, 0), (0, SPAN - EPT))).reshape(-1)
    dst = jnp.pad(edge_index[1].reshape(NSUB, EPT),
                  ((0, 0), (0, SPAN - EPT)),
                  constant_values=-1).reshape(-1)
    zeros = jnp.zeros((RPT, D), jnp.float32)
    x = coord[:, 0:1]
    y = coord[:, 1:2]

    h = _embed(x, y, W_embed, b_embed.reshape(1, D))
    for (W, b) in ((W0, b0), (W1, b1), (W2, b2)):
        agg = _seg_kernel(h, src, dst, zeros)
        h = _update(h, agg, W, b.reshape(1, D))

    out = _score(h, w_score.reshape(D, 1), node_type.reshape(N, 1))
    return out.reshape(N)


# speedup vs baseline: 17.3830x; 1.0009x over previous
import jax
import jax.numpy as jnp
from jax import lax
from jax.experimental import pallas as pl
from jax.experimental.pallas import tpu as pltpu
from jax.experimental.pallas import tpu_sc as plsc

N = 50000
E = 800000
D = 64

NCORES = 2
NSUB = 16
H = N // NCORES
RPT = 1568
HPAD = NSUB * RPT
TRASH = HPAD
ACC_ROWS = HPAD + 8
EPT = E // NSUB

BLK = 2000
NBLK = N // BLK


G = 112
NSLOT = 3
RPB = 6
BGRP = NSLOT * RPB
BCH = BGRP * G
NBATCH = 25
SPAN = NBATCH * BCH


def _make_seg_kernel():
    mesh = plsc.VectorSubcoreMesh(core_axis_name="c", subcore_axis_name="s")

    def body(h_hbm, src_hbm, dst_hbm, zeros_hbm, agg_hbm,
             acc, ibS, ibD, lgrps, rows, semIS, semID, semG, semU):
        c = lax.axis_index("c")
        s = lax.axis_index("s")
        base = c * H
        tbase = s * SPAN

        pltpu.sync_copy(zeros_hbm, acc.at[pl.ds(s * RPT, RPT)])

        def issue_batch_loads(p, b):
            off = tbase + b * BCH
            pltpu.async_copy(src_hbm.at[pl.ds(off, BCH)], ibS[p], semIS[p])
            pltpu.async_copy(dst_hbm.at[pl.ds(off, BCH)], ibD[p], semID[p])

        def wait_scatter(k):
            pltpu.make_async_copy(h_hbm.at[pl.ds(0, G)], rows[k],
                                  semU[k]).wait()

        def issue_gather(bS, k, gl):
            wait_scatter(k)
            pltpu.async_copy(h_hbm.at[bS.at[pl.ds(gl * G, G)]],
                             rows[k], semG[k])

        def slot_cycle(bD, k, gl):
            pltpu.make_async_copy(h_hbm.at[pl.ds(0, G)], rows[k],
                                  semG[k]).wait()
            for i in range(G // 16):
                loc = bD[pl.ds(gl * G + i * 16, 16)] - base
                ok = (loc >= 0) & (loc < H)
                lgrps[k][pl.ds(i * 16, 16)] = jnp.where(ok, loc, TRASH)
            pltpu.async_copy(rows[k], acc.at[lgrps[k]], semU[k], add=True)

        def run_batch(p):
            bS, bD = ibS[p], ibD[p]
            pltpu.make_async_copy(src_hbm.at[pl.ds(0, BCH)], bS,
                                  semIS[p]).wait()
            pltpu.make_async_copy(src_hbm.at[pl.ds(0, BCH)], bD,
                                  semID[p]).wait()
            for k in range(NSLOT):
                issue_gather(bS, k, k)

            def round_(rr, carry):
                for k in range(NSLOT):
                    slot_cycle(bD, k, rr * NSLOT + k)
                for k in range(NSLOT):
                    issue_gather(bS, k, (rr + 1) * NSLOT + k)
                return carry

            lax.fori_loop(0, RPB - 1, round_, 0)
            for k in range(NSLOT):
                slot_cycle(bD, k, (RPB - 1) * NSLOT + k)

        for k in range(NSLOT):
            for i in range(G // 16):
                lgrps[k][pl.ds(i * 16, 16)] = jnp.full((16,), TRASH,
                                                       jnp.int32)
            pltpu.async_copy(rows[k], acc.at[lgrps[k]], semU[k], add=True)

        issue_batch_loads(0, 0)
        issue_batch_loads(1, 1)
        plsc.subcore_barrier()

        def batch(b, carry):
            @pl.when(b % 2 == 0)
            def _():
                run_batch(0)

            @pl.when(b % 2 == 1)
            def _():
                run_batch(1)

            @pl.when(b + 2 < NBATCH)
            def _():
                @pl.when(b % 2 == 0)
                def _():
                    issue_batch_loads(0, b + 2)

                @pl.when(b % 2 == 1)
                def _():
                    issue_batch_loads(1, b + 2)

            return carry

        lax.fori_loop(0, NBATCH, batch, 0)

        for k in range(NSLOT):
            wait_scatter(k)
        plsc.subcore_barrier()

        loff = jnp.minimum(s * RPT, H - RPT)
        pltpu.sync_copy(acc.at[pl.ds(loff, RPT)],
                        agg_hbm.at[pl.ds(base + loff, RPT)])

    return pl.kernel(
        body,
        out_type=jax.ShapeDtypeStruct((N, D), jnp.float32),
        mesh=mesh,
        compiler_params=pltpu.CompilerParams(use_tc_tiling_on_sc=False),
        scratch_types=[
            pltpu.VMEM_SHARED((ACC_ROWS, D), jnp.float32),
            [pltpu.VMEM((BCH,), jnp.int32) for _ in range(2)],
            [pltpu.VMEM((BCH,), jnp.int32) for _ in range(2)],
            [pltpu.VMEM((G,), jnp.int32) for _ in range(NSLOT)],
            [pltpu.VMEM((G, D), jnp.float32) for _ in range(NSLOT)],
            [pltpu.SemaphoreType.DMA for _ in range(2)],
            [pltpu.SemaphoreType.DMA for _ in range(2)],
            [pltpu.SemaphoreType.DMA for _ in range(NSLOT)],
            [pltpu.SemaphoreType.DMA for _ in range(NSLOT)],
        ],
    )


_seg_kernel = _make_seg_kernel()


def _embed_body(x_ref, y_ref, we_ref, b_ref, o_ref):
    o_ref[...] = (x_ref[...] * we_ref[0:1, :]
                  + y_ref[...] * we_ref[1:2, :] + b_ref[...])


def _embed(x, y, W_embed, b):
    return pl.pallas_call(
        _embed_body,
        grid=(NBLK,),
        in_specs=[
            pl.BlockSpec((BLK, 1), lambda i: (i, 0)),
            pl.BlockSpec((BLK, 1), lambda i: (i, 0)),
            pl.BlockSpec((2, D), lambda i: (0, 0)),
            pl.BlockSpec((1, D), lambda i: (0, 0)),
        ],
        out_specs=pl.BlockSpec((BLK, D), lambda i: (i, 0)),
        out_shape=jax.ShapeDtypeStruct((N, D), jnp.float32),
    )(x, y, W_embed, b)


def _update_body(h_ref, agg_ref, w_ref, b_ref, o_ref):
    z = jnp.dot(agg_ref[...], w_ref[...],
                preferred_element_type=jnp.float32) + b_ref[...]
    o_ref[...] = h_ref[...] + jnp.maximum(z, 0.0)


def _update(h, agg, W, b):
    return pl.pallas_call(
        _update_body,
        grid=(NBLK,),
        in_specs=[
            pl.BlockSpec((BLK, D), lambda i: (i, 0)),
            pl.BlockSpec((BLK, D), lambda i: (i, 0)),
            pl.BlockSpec((D, D), lambda i: (0, 0)),
            pl.BlockSpec((1, D), lambda i: (0, 0)),
        ],
        out_specs=pl.BlockSpec((BLK, D), lambda i: (i, 0)),
        out_shape=jax.ShapeDtypeStruct((N, D), jnp.float32),
    )(h, agg, W, b)


def _score_body(h_ref, w_ref, nt_ref, o_ref, msk_s, m_s, s_s):
    i = pl.program_id(0)

    @pl.when(i < NBLK)
    def _():
        sc = jnp.dot(h_ref[...], w_ref[...],
                     preferred_element_type=jnp.float32)
        masked = jnp.where(nt_ref[...] == 2, sc, jnp.float32(-1e9))
        msk_s[pl.ds((i % NBLK) * BLK, BLK), :] = masked

        @pl.when(i == 0)
        def _():
            m_s[0] = jnp.float32(-1e30)
            s_s[0] = jnp.float32(0.0)

        m_old = m_s[0]
        m_new = jnp.maximum(m_old, jnp.max(masked))
        s_s[0] = (s_s[0] * jnp.exp(m_old - m_new)
                  + jnp.sum(jnp.exp(masked - m_new)))
        m_s[0] = m_new

    @pl.when(i >= NBLK)
    def _():
        lse = m_s[0] + jnp.log(s_s[0])
        o_ref[...] = msk_s[pl.ds((i % NBLK) * BLK, BLK), :] - lse


def _score(h, w_score2d, node_type2d):
    return pl.pallas_call(
        _score_body,
        grid=(2 * NBLK,),
        in_specs=[
            pl.BlockSpec((BLK, D), lambda i: (i % NBLK, 0)),
            pl.BlockSpec((D, 1), lambda i: (0, 0)),
            pl.BlockSpec((BLK, 1), lambda i: (i % NBLK, 0)),
        ],
        out_specs=pl.BlockSpec((BLK, 1), lambda i: (i % NBLK, 0)),
        out_shape=jax.ShapeDtypeStruct((N, 1), jnp.float32),
        scratch_shapes=[
            pltpu.VMEM((N, 1), jnp.float32),
            pltpu.SMEM((1,), jnp.float32),
            pltpu.SMEM((1,), jnp.float32),
        ],
    )(h, w_score2d, node_type2d)


@jax.jit
def kernel(coord, W_embed, b_embed, W0, b0, W1, b1, W2, b2, w_score,
           edge_index, node_type):
    src = jnp.pad(edge_index[0].reshape(NSUB, EPT),
                  ((0, 0), (0, SPAN - EPT))).reshape(-1)
    dst = jnp.pad(edge_index[1].reshape(NSUB, EPT),
                  ((0, 0), (0, SPAN - EPT)),
                  constant_values=-1).reshape(-1)
    zeros = jnp.zeros((RPT, D), jnp.float32)
    x = coord[:, 0:1]
    y = coord[:, 1:2]

    h = _embed(x, y, W_embed, b_embed.reshape(1, D))
    for (W, b) in ((W0, b0), (W1, b1), (W2, b2)):
        agg = _seg_kernel(h, src, dst, zeros)
        h = _update(h, agg, W, b.reshape(1, D))

    out = _score(h, w_score.reshape(D, 1), node_type.reshape(N, 1))
    return out.reshape(N)
